# baseline (speedup 1.0000x reference)
import functools

import jax
import jax.numpy as jnp
from jax import lax
from jax.experimental import pallas as pl
from jax.experimental.pallas import tpu as pltpu
from jax.experimental.pallas import tpu_sc as plsc

_CH = 128


@functools.lru_cache(maxsize=None)
def _make_sc_gather(B, DU, DV):
    info = plsc.get_sparse_core_info()
    NC, NS = info.num_cores, info.num_subcores
    NW = NC * NS
    bw = B // NW
    nch = bw // _CH
    mesh = plsc.VectorSubcoreMesh(core_axis_name="c", subcore_axis_name="s")

    @functools.partial(
        pl.kernel,
        mesh=mesh,
        compiler_params=pltpu.CompilerParams(use_tc_tiling_on_sc=False),
        out_type=(
            jax.ShapeDtypeStruct((B, DU), jnp.bfloat16),
            jax.ShapeDtypeStruct((B, DV), jnp.bfloat16),
        ),
        scratch_types=[
            pltpu.VMEM((nch, _CH), jnp.int32),
            pltpu.VMEM((nch, _CH), jnp.int32),
            pltpu.VMEM((bw, DU), jnp.bfloat16),
            pltpu.VMEM((bw, DV), jnp.bfloat16),
            pltpu.SemaphoreType.DMA,
            pltpu.SemaphoreType.DMA,
        ],
    )
    def gather_kernel(ut, uid, vt, vid, u_out, v_out,
                      uidx, vidx, urows, vrows, su, sv):
        wid = lax.axis_index("s") * NC + lax.axis_index("c")
        base = wid * bw
        for c in range(nch):
            pltpu.sync_copy(uid.at[pl.ds(base + c * _CH, _CH)], uidx.at[c])
            pltpu.sync_copy(vid.at[pl.ds(base + c * _CH, _CH)], vidx.at[c])
        ucopies = [
            pltpu.async_copy(ut.at[uidx.at[c]], urows.at[pl.ds(c * _CH, _CH)], su)
            for c in range(nch)
        ]
        vcopies = [
            pltpu.async_copy(vt.at[vidx.at[c]], vrows.at[pl.ds(c * _CH, _CH)], sv)
            for c in range(nch)
        ]
        for c in ucopies:
            c.wait()
        pltpu.sync_copy(urows, u_out.at[pl.ds(base, bw)])
        for c in vcopies:
            c.wait()
        pltpu.sync_copy(vrows, v_out.at[pl.ds(base, bw)])

    return gather_kernel


def _mlp_body(u_ref, v_ref, w1u_ref, w1v_ref, b1_ref, wo_ref, bo_ref, o_ref):
    u = u_ref[...].astype(jnp.float32)
    v = v_ref[...].astype(jnp.float32)
    h = jnp.dot(u, w1u_ref[...], preferred_element_type=jnp.float32)
    h = h + jnp.dot(v, w1v_ref[...], preferred_element_type=jnp.float32)
    h = jnp.maximum(h + b1_ref[...], 0.0)
    o_ref[...] = jnp.dot(h, wo_ref[...], preferred_element_type=jnp.float32) + bo_ref[...]


def _mlp(u_emb, v_emb, w1u_t, w1v_t, b1, wout_t, bout):
    B, D = u_emb.shape
    H = w1u_t.shape[1]
    O = wout_t.shape[1]
    blk = 2048
    return pl.pallas_call(
        _mlp_body,
        grid=(B // blk,),
        in_specs=[
            pl.BlockSpec((blk, D), lambda i: (i, 0)),
            pl.BlockSpec((blk, D), lambda i: (i, 0)),
            pl.BlockSpec((D, H), lambda i: (0, 0)),
            pl.BlockSpec((D, H), lambda i: (0, 0)),
            pl.BlockSpec((1, H), lambda i: (0, 0)),
            pl.BlockSpec((H, O), lambda i: (0, 0)),
            pl.BlockSpec((1, O), lambda i: (0, 0)),
        ],
        out_specs=pl.BlockSpec((blk, O), lambda i: (i, 0)),
        out_shape=jax.ShapeDtypeStruct((B, O), jnp.float32),
    )(u_emb, v_emb, w1u_t, w1v_t, b1, wout_t, bout)


def kernel(user_id, video_id, user_table, video_table, W1, b1, Wout, bout):
    B = user_id.shape[0]
    DU = user_table.shape[1]
    DV = video_table.shape[1]
    gather = _make_sc_gather(B, DU, DV)
    u_emb, v_emb = gather(
        user_table.astype(jnp.bfloat16), user_id.astype(jnp.int32),
        video_table.astype(jnp.bfloat16), video_id.astype(jnp.int32),
    )
    w1u_t = W1[:, :DU].T
    w1v_t = W1[:, DU:].T
    return _mlp(u_emb, v_emb, w1u_t, w1v_t, b1[None, :], Wout.T, bout[None, :])


# speedup vs baseline: 1.8891x; 1.8891x over previous
import functools
import math

import jax
import jax.numpy as jnp
from jax import lax
from jax.experimental import pallas as pl
from jax.experimental.pallas import tpu as pltpu
from jax.experimental.pallas import tpu_sc as plsc

_BLKU = 8192
_Q = _BLKU // 4
_CH = 128


def _pack_body(in_ref, o_ref):
    y = in_ref[...].T
    o_ref[...] = jnp.concatenate(
        [y[0:_Q], y[_Q:2 * _Q], y[2 * _Q:3 * _Q], y[3 * _Q:4 * _Q]], axis=1)


@functools.lru_cache(maxsize=None)
def _make_pack(N, D):
    g = math.ceil(N / _BLKU)
    return pl.pallas_call(
        _pack_body,
        grid=(g,),
        in_specs=[pl.BlockSpec((D, _BLKU), lambda i: (0, i))],
        out_specs=pl.BlockSpec((_Q, 128), lambda i: (i, 0)),
        out_shape=jax.ShapeDtypeStruct((g * _Q, 128), jnp.float32),
    )


@functools.lru_cache(maxsize=None)
def _make_sc_gather(B, GU, GV):
    info = plsc.get_sparse_core_info()
    NC, NS = info.num_cores, info.num_subcores
    NW = NC * NS
    bw = B // NW
    nch = bw // _CH
    mesh = plsc.VectorSubcoreMesh(core_axis_name="c", subcore_axis_name="s")

    @functools.partial(
        pl.kernel,
        mesh=mesh,
        compiler_params=pltpu.CompilerParams(
            use_tc_tiling_on_sc=False, needs_layout_passes=False),
        out_type=jax.ShapeDtypeStruct((B, 128), jnp.float32),
        scratch_types=[
            pltpu.VMEM((nch, _CH), jnp.int32),
            pltpu.VMEM((nch, _CH), jnp.int32),
            pltpu.VMEM((nch, _CH), jnp.int32),
            pltpu.VMEM((nch, _CH), jnp.int32),
            pltpu.VMEM((2, _CH, 128), jnp.float32),
            pltpu.VMEM((2, _CH, 128), jnp.float32),
            pltpu.VMEM((2, _CH, 128), jnp.float32),
            pltpu.SemaphoreType.DMA,
            pltpu.SemaphoreType.DMA,
            pltpu.SemaphoreType.DMA,
        ],
    )
    def gather_kernel(upk, uid, vpk, vid, out,
                      upidx, uoff, vpidx, voff, uslab, vslab, cc,
                      sgu, sgv, sw):
        wid = lax.axis_index("s") * NC + lax.axis_index("c")
        base = wid * bw
        iota = lax.iota(jnp.int32, 16)

        for c in range(nch):
            pltpu.sync_copy(uid.at[pl.ds(base + c * _CH, _CH)], upidx.at[c])
            pltpu.sync_copy(vid.at[pl.ds(base + c * _CH, _CH)], vpidx.at[c])

        def idx_body(s, c):
            sl = pl.ds(s * 16, 16)
            for pidx_ref, off_ref in ((upidx, uoff), (vpidx, voff)):
                u = pidx_ref[c, sl]
                off_ref[c, sl] = ((u >> 11) & 3) << 5
                pidx_ref[c, sl] = ((u >> 13) << 11) | (u & 2047)
            return c

        for c in range(nch):
            lax.fori_loop(0, _CH // 16, idx_body, c)

        def zero_body(r, _):
            z = jnp.zeros((16,), jnp.float32)
            for b in range(2):
                for l0 in (64, 80, 96, 112):
                    cc[b, r, pl.ds(l0, 16)] = z
            return 0

        lax.fori_loop(0, _CH, zero_body, 0)

        def issue(c):
            b = c % 2
            gu = pltpu.async_copy(upk.at[upidx.at[c]], uslab.at[b], sgu)
            gv = pltpu.async_copy(vpk.at[vpidx.at[c]], vslab.at[b], sgv)
            return gu, gv

        def extract(c):
            b = c % 2

            def ex_body(s, _):
                rows = s * 16 + iota
                ou = uoff[c, pl.ds(s * 16, 16)]
                ov = voff[c, pl.ds(s * 16, 16)]
                for d in range(32):
                    lane = jnp.full((16,), d, jnp.int32)
                    val = plsc.load_gather(uslab.at[b], [rows, ou + d])
                    plsc.store_scatter(cc.at[b], [rows, lane], val)
                    val = plsc.load_gather(vslab.at[b], [rows, ov + d])
                    plsc.store_scatter(cc.at[b], [rows, lane + 32], val)
                return 0

            lax.fori_loop(0, _CH // 16, ex_body, 0)

        pend = {0: issue(0)}
        writes = []
        for c in range(nch):
            if c + 1 < nch:
                pend[c + 1] = issue(c + 1)
            gu, gv = pend.pop(c)
            gu.wait()
            gv.wait()
            if c >= 2:
                writes[c - 2].wait()
            extract(c)
            writes.append(pltpu.async_copy(
                cc.at[c % 2], out.at[pl.ds(base + c * _CH, _CH)], sw))
        writes[-2].wait()
        writes[-1].wait()

    return gather_kernel


def _mlp_body(x_ref, w1_ref, b1_ref, wo_ref, bo_ref, o_ref):
    h = jnp.dot(x_ref[...], w1_ref[...], preferred_element_type=jnp.float32)
    h = jnp.maximum(h + b1_ref[...], 0.0)
    o_ref[...] = jnp.dot(h, wo_ref[...], preferred_element_type=jnp.float32) + bo_ref[...]


@functools.lru_cache(maxsize=None)
def _make_mlp(B, H, O):
    blk = 2048
    return pl.pallas_call(
        _mlp_body,
        grid=(B // blk,),
        in_specs=[
            pl.BlockSpec((blk, 128), lambda i: (i, 0)),
            pl.BlockSpec((128, H), lambda i: (0, 0)),
            pl.BlockSpec((1, H), lambda i: (0, 0)),
            pl.BlockSpec((H, O), lambda i: (0, 0)),
            pl.BlockSpec((1, O), lambda i: (0, 0)),
        ],
        out_specs=pl.BlockSpec((blk, O), lambda i: (i, 0)),
        out_shape=jax.ShapeDtypeStruct((B, O), jnp.float32),
    )


def kernel(user_id, video_id, user_table, video_table, W1, b1, Wout, bout):
    B = user_id.shape[0]
    NU, D = user_table.shape
    NV = video_table.shape[0]
    upk = _make_pack(NU, D)(user_table.T)
    vpk = _make_pack(NV, D)(video_table.T)
    GU = upk.shape[0] // _Q
    GV = vpk.shape[0] // _Q
    cc = _make_sc_gather(B, GU, GV)(
        upk, user_id.astype(jnp.int32), vpk, video_id.astype(jnp.int32))
    H = W1.shape[0]
    O = Wout.shape[0]
    w1pad = jnp.zeros((128, H), jnp.float32).at[:2 * D].set(W1.T)
    return _make_mlp(B, H, O)(cc, w1pad, b1[None, :], Wout.T, bout[None, :])


# speedup vs baseline: 2.6394x; 1.3972x over previous
import functools
import math

import jax
import jax.numpy as jnp
from jax import lax
from jax.experimental import pallas as pl
from jax.experimental.pallas import tpu as pltpu
from jax.experimental.pallas import tpu_sc as plsc

_BLKU = 8192
_Q = _BLKU // 4
_CH = 128


def _pack_body(in_ref, o_ref):
    x = in_ref[...]
    d = x.shape[0]
    rows = lax.broadcasted_iota(jnp.int32, (d, 4 * d), 0)
    cols = lax.broadcasted_iota(jnp.int32, (d, 4 * d), 1)
    acc = None
    for j in range(4):
        xj = x[:, j * _Q:(j + 1) * _Q]
        ej = jnp.where(cols == rows + d * j, 1.0, 0.0).astype(jnp.bfloat16)
        y = lax.dot_general(xj.astype(jnp.bfloat16), ej,
                            (((0,), (0,)), ((), ())),
                            preferred_element_type=jnp.float32)
        acc = y if acc is None else acc + y
    o_ref[...] = acc


@functools.lru_cache(maxsize=None)
def _make_pack(N, D):
    g = math.ceil(N / _BLKU)
    return pl.pallas_call(
        _pack_body,
        grid=(g,),
        compiler_params=pltpu.CompilerParams(
            fuse_transposed_lhs_in_matmul=True),
        in_specs=[pl.BlockSpec((D, _BLKU), lambda i: (0, i))],
        out_specs=pl.BlockSpec((_Q, 128), lambda i: (i, 0)),
        out_shape=jax.ShapeDtypeStruct((g * _Q, 128), jnp.float32),
    )


@functools.lru_cache(maxsize=None)
def _make_sc_gather(B, GU, GV):
    info = plsc.get_sparse_core_info()
    NC, NS = info.num_cores, info.num_subcores
    NW = NC * NS
    bw = B // NW
    nch = bw // _CH
    mesh = plsc.VectorSubcoreMesh(core_axis_name="c", subcore_axis_name="s")

    @functools.partial(
        pl.kernel,
        mesh=mesh,
        compiler_params=pltpu.CompilerParams(
            use_tc_tiling_on_sc=False, needs_layout_passes=False),
        out_type=jax.ShapeDtypeStruct((B, 128), jnp.float32),
        scratch_types=[
            pltpu.VMEM((nch, _CH), jnp.int32),
            pltpu.VMEM((nch, _CH), jnp.int32),
            pltpu.VMEM((nch, _CH), jnp.int32),
            pltpu.VMEM((nch, _CH), jnp.int32),
            pltpu.VMEM((2, _CH, 128), jnp.float32),
            pltpu.VMEM((2, _CH, 128), jnp.float32),
            pltpu.VMEM((2, _CH, 128), jnp.float32),
            pltpu.SemaphoreType.DMA,
            pltpu.SemaphoreType.DMA,
            pltpu.SemaphoreType.DMA,
        ],
    )
    def gather_kernel(upk, uid, vpk, vid, out,
                      upidx, uoff, vpidx, voff, uslab, vslab, cc,
                      sgu, sgv, sw):
        wid = lax.axis_index("s") * NC + lax.axis_index("c")
        base = wid * bw
        iota = lax.iota(jnp.int32, 16)

        for c in range(nch):
            pltpu.sync_copy(uid.at[pl.ds(base + c * _CH, _CH)], upidx.at[c])
            pltpu.sync_copy(vid.at[pl.ds(base + c * _CH, _CH)], vpidx.at[c])

        def idx_body(s, c):
            sl = pl.ds(s * 16, 16)
            for pidx_ref, off_ref in ((upidx, uoff), (vpidx, voff)):
                u = pidx_ref[c, sl]
                off_ref[c, sl] = ((u >> 11) & 3) << 5
                pidx_ref[c, sl] = ((u >> 13) << 11) | (u & 2047)
            return c

        for c in range(nch):
            lax.fori_loop(0, _CH // 16, idx_body, c)

        def zero_body(r, _):
            z = jnp.zeros((16,), jnp.float32)
            for b in range(2):
                for l0 in (64, 80, 96, 112):
                    cc[b, r, pl.ds(l0, 16)] = z
            return 0

        lax.fori_loop(0, _CH, zero_body, 0)

        def issue(c):
            b = c % 2
            gu = pltpu.async_copy(upk.at[upidx.at[c]], uslab.at[b], sgu)
            gv = pltpu.async_copy(vpk.at[vpidx.at[c]], vslab.at[b], sgv)
            return gu, gv

        def extract(c):
            b = c % 2

            def ex_body(s, _):
                rows = s * 16 + iota
                ou = uoff[c, pl.ds(s * 16, 16)]
                ov = voff[c, pl.ds(s * 16, 16)]
                for d in range(32):
                    lane = jnp.full((16,), d, jnp.int32)
                    val = plsc.load_gather(uslab.at[b], [rows, ou + d])
                    plsc.store_scatter(cc.at[b], [rows, lane], val)
                    val = plsc.load_gather(vslab.at[b], [rows, ov + d])
                    plsc.store_scatter(cc.at[b], [rows, lane + 32], val)
                return 0

            lax.fori_loop(0, _CH // 16, ex_body, 0)

        pend = {0: issue(0)}
        writes = []
        for c in range(nch):
            if c + 1 < nch:
                pend[c + 1] = issue(c + 1)
            gu, gv = pend.pop(c)
            gu.wait()
            gv.wait()
            if c >= 2:
                writes[c - 2].wait()
            extract(c)
            writes.append(pltpu.async_copy(
                cc.at[c % 2], out.at[pl.ds(base + c * _CH, _CH)], sw))
        writes[-2].wait()
        writes[-1].wait()

    return gather_kernel


def _mlp_body(x_ref, w1_ref, b1_ref, wo_ref, bo_ref, o_ref):
    h = jnp.dot(x_ref[...], w1_ref[...], preferred_element_type=jnp.float32)
    h = jnp.maximum(h + b1_ref[...], 0.0)
    o_ref[...] = jnp.dot(h, wo_ref[...], preferred_element_type=jnp.float32) + bo_ref[...]


@functools.lru_cache(maxsize=None)
def _make_mlp(B, H, O):
    blk = 2048
    return pl.pallas_call(
        _mlp_body,
        grid=(B // blk,),
        in_specs=[
            pl.BlockSpec((blk, 128), lambda i: (i, 0)),
            pl.BlockSpec((128, H), lambda i: (0, 0)),
            pl.BlockSpec((1, H), lambda i: (0, 0)),
            pl.BlockSpec((H, O), lambda i: (0, 0)),
            pl.BlockSpec((1, O), lambda i: (0, 0)),
        ],
        out_specs=pl.BlockSpec((blk, O), lambda i: (i, 0)),
        out_shape=jax.ShapeDtypeStruct((B, O), jnp.float32),
    )


def kernel(user_id, video_id, user_table, video_table, W1, b1, Wout, bout):
    B = user_id.shape[0]
    NU, D = user_table.shape
    NV = video_table.shape[0]
    upk = _make_pack(NU, D)(user_table.T)
    vpk = _make_pack(NV, D)(video_table.T)
    GU = upk.shape[0] // _Q
    GV = vpk.shape[0] // _Q
    cc = _make_sc_gather(B, GU, GV)(
        upk, user_id.astype(jnp.int32), vpk, video_id.astype(jnp.int32))
    H = W1.shape[0]
    O = Wout.shape[0]
    w1pad = jnp.zeros((128, H), jnp.float32).at[:2 * D].set(W1.T)
    return _make_mlp(B, H, O)(cc, w1pad, b1[None, :], Wout.T, bout[None, :])


# speedup vs baseline: 2.6860x; 1.0177x over previous
import functools
import math

import jax
import jax.numpy as jnp
from jax import lax
from jax.experimental import pallas as pl
from jax.experimental.pallas import tpu as pltpu
from jax.experimental.pallas import tpu_sc as plsc

_BLKU = 8192
_Q = _BLKU // 4
_CH = 128


def _pack_body(in_ref, o_ref):
    x = in_ref[...]
    d = x.shape[0]
    rows = lax.broadcasted_iota(jnp.int32, (d, 4 * d), 0)
    cols = lax.broadcasted_iota(jnp.int32, (d, 4 * d), 1)
    acc = None
    for j in range(4):
        xj = x[:, j * _Q:(j + 1) * _Q]
        ej = jnp.where(cols == rows + d * j, 1.0, 0.0).astype(jnp.bfloat16)
        y = lax.dot_general(xj.astype(jnp.bfloat16), ej,
                            (((0,), (0,)), ((), ())),
                            preferred_element_type=jnp.float32)
        acc = y if acc is None else acc + y
    o_ref[...] = acc


@functools.lru_cache(maxsize=None)
def _make_pack(N, D):
    g = math.ceil(N / _BLKU)
    return pl.pallas_call(
        _pack_body,
        grid=(g,),
        in_specs=[pl.BlockSpec((D, _BLKU), lambda i: (0, i))],
        out_specs=pl.BlockSpec((_Q, 128), lambda i: (i, 0)),
        out_shape=jax.ShapeDtypeStruct((g * _Q, 128), jnp.float32),
    )


@functools.lru_cache(maxsize=None)
def _make_sc_gather(B, G):
    info = plsc.get_sparse_core_info()
    NC, NS = info.num_cores, info.num_subcores
    NW = NC * NS
    bw = B // NW
    nch = bw // _CH
    mesh = plsc.VectorSubcoreMesh(core_axis_name="c", subcore_axis_name="s")

    @functools.partial(
        pl.kernel,
        mesh=mesh,
        compiler_params=pltpu.CompilerParams(
            use_tc_tiling_on_sc=False, needs_layout_passes=False),
        out_type=jax.ShapeDtypeStruct((B, 128), jnp.float32),
        scratch_types=[
            pltpu.VMEM((nch, _CH), jnp.int32),
            pltpu.VMEM((nch, _CH), jnp.int32),
            pltpu.VMEM((2, _CH, 128), jnp.float32),
            pltpu.VMEM((2, _CH, 128), jnp.float32),
            pltpu.SemaphoreType.DMA,
            pltpu.SemaphoreType.DMA,
        ],
    )
    def gather_kernel(pk, ids, out, pidx, off, slab, cc, sg, sw):
        wid = lax.axis_index("s") * NC + lax.axis_index("c")
        base = wid * bw
        iota = lax.iota(jnp.int32, 16)

        for c in range(nch):
            pltpu.sync_copy(ids.at[pl.ds(base + c * _CH, _CH)], pidx.at[c])

        def idx_body(s, c):
            sl = pl.ds(s * 16, 16)
            u = pidx[c, sl]
            off[c, sl] = ((u >> 11) & 3) << 5
            pidx[c, sl] = ((u >> 13) << 11) | (u & 2047)
            return c

        for c in range(nch):
            lax.fori_loop(0, _CH // 16, idx_body, c)

        def zero_body(r, _):
            z = jnp.zeros((16,), jnp.float32)
            for b in range(2):
                for l0 in range(32, 128, 16):
                    cc[b, r, pl.ds(l0, 16)] = z
            return 0

        lax.fori_loop(0, _CH, zero_body, 0)

        def issue(c):
            return pltpu.async_copy(pk.at[pidx.at[c]], slab.at[c % 2], sg)

        def extract(c):
            b = c % 2

            def ex_body(s, _):
                rows = s * 16 + iota
                o = off[c, pl.ds(s * 16, 16)]
                for d in range(32):
                    lane = jnp.full((16,), d, jnp.int32)
                    val = plsc.load_gather(slab.at[b], [rows, o + d])
                    plsc.store_scatter(cc.at[b], [rows, lane], val)
                return 0

            lax.fori_loop(0, _CH // 16, ex_body, 0)

        pend = {0: issue(0)}
        writes = []
        for c in range(nch):
            if c + 1 < nch:
                pend[c + 1] = issue(c + 1)
            pend.pop(c).wait()
            if c >= 2:
                writes[c - 2].wait()
            extract(c)
            writes.append(pltpu.async_copy(
                cc.at[c % 2], out.at[pl.ds(base + c * _CH, _CH)], sw))
        writes[-2].wait()
        writes[-1].wait()

    return gather_kernel


def _mlp_body(u_ref, v_ref, wu_ref, wv_ref, b1_ref, wo_ref, bo_ref, o_ref):
    h = jnp.dot(u_ref[...], wu_ref[...], preferred_element_type=jnp.float32)
    h = h + jnp.dot(v_ref[...], wv_ref[...], preferred_element_type=jnp.float32)
    h = jnp.maximum(h + b1_ref[...], 0.0)
    o_ref[...] = jnp.dot(h, wo_ref[...], preferred_element_type=jnp.float32) + bo_ref[...]


@functools.lru_cache(maxsize=None)
def _make_mlp(B, H, O):
    blk = 2048
    return pl.pallas_call(
        _mlp_body,
        grid=(B // blk,),
        in_specs=[
            pl.BlockSpec((blk, 128), lambda i: (i, 0)),
            pl.BlockSpec((blk, 128), lambda i: (i, 0)),
            pl.BlockSpec((128, H), lambda i: (0, 0)),
            pl.BlockSpec((128, H), lambda i: (0, 0)),
            pl.BlockSpec((1, H), lambda i: (0, 0)),
            pl.BlockSpec((H, O), lambda i: (0, 0)),
            pl.BlockSpec((1, O), lambda i: (0, 0)),
        ],
        out_specs=pl.BlockSpec((blk, O), lambda i: (i, 0)),
        out_shape=jax.ShapeDtypeStruct((B, O), jnp.float32),
    )


def kernel(user_id, video_id, user_table, video_table, W1, b1, Wout, bout):
    B = user_id.shape[0]
    NU, D = user_table.shape
    NV = video_table.shape[0]
    vpk = _make_pack(NV, D)(video_table.T)
    ccv = _make_sc_gather(B, vpk.shape[0] // _Q)(vpk, video_id.astype(jnp.int32))
    upk = _make_pack(NU, D)(user_table.T)
    ccu = _make_sc_gather(B, upk.shape[0] // _Q)(upk, user_id.astype(jnp.int32))
    H = W1.shape[0]
    O = Wout.shape[0]
    wu = jnp.zeros((128, H), jnp.float32).at[:D].set(W1[:, :D].T)
    wv = jnp.zeros((128, H), jnp.float32).at[:D].set(W1[:, D:].T)
    return _make_mlp(B, H, O)(ccu, ccv, wu, wv, b1[None, :], Wout.T, bout[None, :])


# speedup vs baseline: 2.8019x; 1.0431x over previous
import functools
import math

import jax
import jax.numpy as jnp
from jax import lax
from jax.experimental import pallas as pl
from jax.experimental.pallas import tpu as pltpu
from jax.experimental.pallas import tpu_sc as plsc

_BLKU = 8192
_Q = _BLKU // 4
_CH = 128


def _pack_body(in_ref, o_ref):
    x = in_ref[...]
    d = x.shape[0]
    rows = lax.broadcasted_iota(jnp.int32, (d, 4 * d), 0)
    cols = lax.broadcasted_iota(jnp.int32, (d, 4 * d), 1)
    acc = None
    for j in range(4):
        xj = x[:, j * _Q:(j + 1) * _Q]
        ej = jnp.where(cols == rows + d * j, 1.0, 0.0).astype(jnp.bfloat16)
        y = lax.dot_general(xj.astype(jnp.bfloat16), ej,
                            (((0,), (0,)), ((), ())),
                            preferred_element_type=jnp.float32)
        acc = y if acc is None else acc + y
    o_ref[...] = acc


@functools.lru_cache(maxsize=None)
def _make_pack(N, D):
    g = math.ceil(N / _BLKU)
    return pl.pallas_call(
        _pack_body,
        grid=(g,),
        in_specs=[pl.BlockSpec((D, _BLKU), lambda i: (0, i))],
        out_specs=pl.BlockSpec((_Q, 128), lambda i: (i, 0)),
        out_shape=jax.ShapeDtypeStruct((g * _Q, 128), jnp.float32),
    )


@functools.lru_cache(maxsize=None)
def _make_sc_gather(B, G):
    info = plsc.get_sparse_core_info()
    NC, NS = info.num_cores, info.num_subcores
    NW = NC * NS
    bw = B // NW
    nch = bw // _CH
    mesh = plsc.VectorSubcoreMesh(core_axis_name="c", subcore_axis_name="s")

    @functools.partial(
        pl.kernel,
        mesh=mesh,
        compiler_params=pltpu.CompilerParams(
            use_tc_tiling_on_sc=False, needs_layout_passes=False),
        out_type=jax.ShapeDtypeStruct((B, 128), jnp.float32),
        scratch_types=[
            pltpu.VMEM((nch, _CH), jnp.int32),
            pltpu.VMEM((nch, _CH), jnp.int32),
            pltpu.VMEM((2, _CH, 128), jnp.float32),
            pltpu.VMEM((2, _CH, 128), jnp.float32),
            pltpu.SemaphoreType.DMA,
            pltpu.SemaphoreType.DMA,
        ],
    )
    def gather_kernel(pk, ids, out, pidx, off, slab, cc, sg, sw):
        wid = lax.axis_index("s") * NC + lax.axis_index("c")
        base = wid * bw
        iota = lax.iota(jnp.int32, 16)

        for c in range(nch):
            pltpu.sync_copy(ids.at[pl.ds(base + c * _CH, _CH)], pidx.at[c])

        def idx_body(s, c):
            sl = pl.ds(s * 16, 16)
            u = pidx[c, sl]
            off[c, sl] = ((u >> 11) & 3) << 5
            pidx[c, sl] = ((u >> 13) << 11) | (u & 2047)
            return c

        for c in range(nch):
            lax.fori_loop(0, _CH // 16, idx_body, c)

        def zero_body(r, _):
            z = jnp.zeros((16,), jnp.float32)
            for b in range(2):
                for l0 in range(32, 128, 16):
                    cc[b, r, pl.ds(l0, 16)] = z
            return 0

        lax.fori_loop(0, _CH, zero_body, 0)

        def issue(c):
            return pltpu.async_copy(pk.at[pidx.at[c]], slab.at[c % 2], sg)

        def extract(c):
            b = c % 2

            def ex_body(s, _):
                rows = s * 16 + iota
                o = off[c, pl.ds(s * 16, 16)]
                for d in range(32):
                    lane = jnp.full((16,), d, jnp.int32)
                    val = plsc.load_gather(slab.at[b], [rows, o + d])
                    plsc.store_scatter(cc.at[b], [rows, lane], val)
                return 0

            lax.fori_loop(0, _CH // 16, ex_body, 0)

        pend = {0: issue(0)}
        writes = []
        for c in range(nch):
            if c + 1 < nch:
                pend[c + 1] = issue(c + 1)
            pend.pop(c).wait()
            if c >= 2:
                writes[c - 2].wait()
            extract(c)
            writes.append(pltpu.async_copy(
                cc.at[c % 2], out.at[pl.ds(base + c * _CH, _CH)], sw))
        writes[-2].wait()
        writes[-1].wait()

    return gather_kernel


def _mlp_body(u_ref, v_ref, wu_ref, wv_ref, b1_ref, wo_ref, bo_ref, o_ref):
    h = jnp.dot(u_ref[...], wu_ref[...], preferred_element_type=jnp.float32)
    h = h + jnp.dot(v_ref[...], wv_ref[...], preferred_element_type=jnp.float32)
    h = jnp.maximum(h + b1_ref[...], 0.0)
    o_ref[...] = jnp.dot(h, wo_ref[...], preferred_element_type=jnp.float32) + bo_ref[...]


@functools.lru_cache(maxsize=None)
def _make_mlp(B, H, O):
    blk = 2048
    return pl.pallas_call(
        _mlp_body,
        grid=(B // blk,),
        in_specs=[
            pl.BlockSpec((blk, 128), lambda i: (i, 0)),
            pl.BlockSpec((blk, 128), lambda i: (i, 0)),
            pl.BlockSpec((128, H), lambda i: (0, 0)),
            pl.BlockSpec((128, H), lambda i: (0, 0)),
            pl.BlockSpec((1, H), lambda i: (0, 0)),
            pl.BlockSpec((H, O), lambda i: (0, 0)),
            pl.BlockSpec((1, O), lambda i: (0, 0)),
        ],
        out_specs=pl.BlockSpec((blk, O), lambda i: (i, 0)),
        out_shape=jax.ShapeDtypeStruct((B, O), jnp.float32),
    )


def kernel(user_id, video_id, user_table, video_table, W1, b1, Wout, bout):
    B = user_id.shape[0]
    NU, D = user_table.shape
    NV = video_table.shape[0]
    vpk = _make_pack(NV, D)(video_table.T)
    ut, vpk = lax.optimization_barrier((user_table, vpk))
    ccv = _make_sc_gather(B, vpk.shape[0] // _Q)(vpk, video_id.astype(jnp.int32))
    upk = _make_pack(NU, D)(ut.T)
    ccu = _make_sc_gather(B, upk.shape[0] // _Q)(upk, user_id.astype(jnp.int32))
    H = W1.shape[0]
    O = Wout.shape[0]
    wu = jnp.zeros((128, H), jnp.float32).at[:D].set(W1[:, :D].T)
    wv = jnp.zeros((128, H), jnp.float32).at[:D].set(W1[:, D:].T)
    return _make_mlp(B, H, O)(ccu, ccv, wu, wv, b1[None, :], Wout.T, bout[None, :])


# speedup vs baseline: 3.3484x; 1.1950x over previous
import functools
import math

import jax
import jax.numpy as jnp
from jax import lax
from jax.experimental import pallas as pl
from jax.experimental.pallas import tpu as pltpu
from jax.experimental.pallas import tpu_sc as plsc

_BLKU = 16384
_Q = _BLKU // 4
_CH = 128


def _pack_body(in_ref, o_ref):
    x = in_ref[...]
    d = x.shape[0]
    rows = lax.broadcasted_iota(jnp.int32, (d, 4 * d), 0)
    cols = lax.broadcasted_iota(jnp.int32, (d, 4 * d), 1)
    acc = None
    for j in range(4):
        xj = x[:, j * _Q:(j + 1) * _Q]
        ej = jnp.where(cols == rows + d * j, 1.0, 0.0).astype(jnp.bfloat16)
        y = lax.dot_general(xj.astype(jnp.bfloat16), ej,
                            (((0,), (0,)), ((), ())),
                            preferred_element_type=jnp.float32)
        acc = y if acc is None else acc + y
    o_ref[...] = acc


@functools.lru_cache(maxsize=None)
def _make_pack(N, D):
    g = math.ceil(N / _BLKU)
    return pl.pallas_call(
        _pack_body,
        grid=(g,),
        in_specs=[pl.BlockSpec((D, _BLKU), lambda i: (0, i))],
        out_specs=pl.BlockSpec((_Q, 128), lambda i: (i, 0)),
        out_shape=jax.ShapeDtypeStruct((g * _Q, 128), jnp.float32),
    )


@functools.lru_cache(maxsize=None)
def _make_sc_gather(B, G):
    info = plsc.get_sparse_core_info()
    NC, NS = info.num_cores, info.num_subcores
    NW = NC * NS
    bw = B // NW
    nch = bw // _CH
    mesh = plsc.VectorSubcoreMesh(core_axis_name="c", subcore_axis_name="s")

    @functools.partial(
        pl.kernel,
        mesh=mesh,
        compiler_params=pltpu.CompilerParams(
            use_tc_tiling_on_sc=False, needs_layout_passes=False),
        out_type=jax.ShapeDtypeStruct((B, 128), jnp.float32),
        scratch_types=[
            pltpu.VMEM((nch, _CH), jnp.int32),
            pltpu.VMEM((nch, _CH), jnp.int32),
            pltpu.VMEM((2, _CH, 128), jnp.float32),
            pltpu.VMEM((2, _CH, 128), jnp.float32),
            pltpu.SemaphoreType.DMA,
            pltpu.SemaphoreType.DMA,
        ],
    )
    def gather_kernel(pk, ids, out, pidx, off, slab, cc, sg, sw):
        wid = lax.axis_index("s") * NC + lax.axis_index("c")
        base = wid * bw
        iota = lax.iota(jnp.int32, 16)

        for c in range(nch):
            pltpu.sync_copy(ids.at[pl.ds(base + c * _CH, _CH)], pidx.at[c])

        qb = _Q.bit_length() - 1
        bb = qb + 2

        def idx_body(s, c):
            sl = pl.ds(s * 16, 16)
            u = pidx[c, sl]
            off[c, sl] = ((u >> qb) & 3) << 5
            pidx[c, sl] = ((u >> bb) << qb) | (u & (_Q - 1))
            return c

        for c in range(nch):
            lax.fori_loop(0, _CH // 16, idx_body, c)

        def zero_body(r, _):
            z = jnp.zeros((16,), jnp.float32)
            for b in range(2):
                for l0 in range(32, 128, 16):
                    cc[b, r, pl.ds(l0, 16)] = z
            return 0

        lax.fori_loop(0, _CH, zero_body, 0)

        def issue(c):
            return pltpu.async_copy(pk.at[pidx.at[c]], slab.at[c % 2], sg)

        def extract(c):
            b = c % 2

            def ex_body(s, _):
                rows = s * 16 + iota
                o = off[c, pl.ds(s * 16, 16)]
                for d in range(32):
                    lane = jnp.full((16,), d, jnp.int32)
                    val = plsc.load_gather(slab.at[b], [rows, o + d])
                    plsc.store_scatter(cc.at[b], [rows, lane], val)
                return 0

            lax.fori_loop(0, _CH // 16, ex_body, 0)

        pend = {0: issue(0)}
        writes = []
        for c in range(nch):
            if c + 1 < nch:
                pend[c + 1] = issue(c + 1)
            pend.pop(c).wait()
            if c >= 2:
                writes[c - 2].wait()
            extract(c)
            writes.append(pltpu.async_copy(
                cc.at[c % 2], out.at[pl.ds(base + c * _CH, _CH)], sw))
        writes[-2].wait()
        writes[-1].wait()

    return gather_kernel


def _mlp_body(u_ref, v_ref, wu_ref, wv_ref, b1_ref, wo_ref, bo_ref, o_ref):
    h = jnp.dot(u_ref[...], wu_ref[...], preferred_element_type=jnp.float32)
    h = h + jnp.dot(v_ref[...], wv_ref[...], preferred_element_type=jnp.float32)
    h = jnp.maximum(h + b1_ref[...], 0.0)
    o_ref[...] = jnp.dot(h, wo_ref[...], preferred_element_type=jnp.float32) + bo_ref[...]


@functools.lru_cache(maxsize=None)
def _make_mlp(B, H, O):
    blk = 2048
    return pl.pallas_call(
        _mlp_body,
        grid=(B // blk,),
        in_specs=[
            pl.BlockSpec((blk, 128), lambda i: (i, 0)),
            pl.BlockSpec((blk, 128), lambda i: (i, 0)),
            pl.BlockSpec((128, H), lambda i: (0, 0)),
            pl.BlockSpec((128, H), lambda i: (0, 0)),
            pl.BlockSpec((1, H), lambda i: (0, 0)),
            pl.BlockSpec((H, O), lambda i: (0, 0)),
            pl.BlockSpec((1, O), lambda i: (0, 0)),
        ],
        out_specs=pl.BlockSpec((blk, O), lambda i: (i, 0)),
        out_shape=jax.ShapeDtypeStruct((B, O), jnp.float32),
    )


def kernel(user_id, video_id, user_table, video_table, W1, b1, Wout, bout):
    B = user_id.shape[0]
    NU, D = user_table.shape
    NV = video_table.shape[0]
    vpk = _make_pack(NV, D)(video_table.T)
    ut, vpk = lax.optimization_barrier((user_table, vpk))
    ccv = _make_sc_gather(B, vpk.shape[0] // _Q)(vpk, video_id.astype(jnp.int32))
    upk = _make_pack(NU, D)(ut.T)
    ccu = _make_sc_gather(B, upk.shape[0] // _Q)(upk, user_id.astype(jnp.int32))
    H = W1.shape[0]
    O = Wout.shape[0]
    wu = jnp.zeros((128, H), jnp.float32).at[:D].set(W1[:, :D].T)
    wv = jnp.zeros((128, H), jnp.float32).at[:D].set(W1[:, D:].T)
    return _make_mlp(B, H, O)(ccu, ccv, wu, wv, b1[None, :], Wout.T, bout[None, :])


# speedup vs baseline: 3.6879x; 1.1014x over previous
import functools
import math

import jax
import jax.numpy as jnp
from jax import lax
from jax.experimental import pallas as pl
from jax.experimental.pallas import tpu as pltpu
from jax.experimental.pallas import tpu_sc as plsc

_BLKU = 32768
_Q = _BLKU // 4
_CH = 128


def _pack_body(in_ref, o_ref):
    x = in_ref[...]
    d = x.shape[0]
    rows = lax.broadcasted_iota(jnp.int32, (d, 4 * d), 0)
    cols = lax.broadcasted_iota(jnp.int32, (d, 4 * d), 1)
    acc = None
    for j in range(4):
        xj = x[:, j * _Q:(j + 1) * _Q]
        ej = jnp.where(cols == rows + d * j, 1.0, 0.0).astype(jnp.bfloat16)
        y = lax.dot_general(xj.astype(jnp.bfloat16), ej,
                            (((0,), (0,)), ((), ())),
                            preferred_element_type=jnp.float32)
        acc = y if acc is None else acc + y
    o_ref[...] = acc


@functools.lru_cache(maxsize=None)
def _make_pack(N, D):
    g = math.ceil(N / _BLKU)
    return pl.pallas_call(
        _pack_body,
        grid=(g,),
        in_specs=[pl.BlockSpec((D, _BLKU), lambda i: (0, i))],
        out_specs=pl.BlockSpec((_Q, 128), lambda i: (i, 0)),
        out_shape=jax.ShapeDtypeStruct((g * _Q, 128), jnp.float32),
    )


@functools.lru_cache(maxsize=None)
def _make_sc_gather(B, G):
    info = plsc.get_sparse_core_info()
    NC, NS = info.num_cores, info.num_subcores
    NW = NC * NS
    bw = B // NW
    nch = bw // _CH
    mesh = plsc.VectorSubcoreMesh(core_axis_name="c", subcore_axis_name="s")

    @functools.partial(
        pl.kernel,
        mesh=mesh,
        compiler_params=pltpu.CompilerParams(
            use_tc_tiling_on_sc=False, needs_layout_passes=False),
        out_type=jax.ShapeDtypeStruct((B, 128), jnp.float32),
        scratch_types=[
            pltpu.VMEM((nch, _CH), jnp.int32),
            pltpu.VMEM((nch, _CH), jnp.int32),
            pltpu.VMEM((2, _CH, 128), jnp.float32),
            pltpu.VMEM((2, _CH, 128), jnp.float32),
            pltpu.SemaphoreType.DMA,
            pltpu.SemaphoreType.DMA,
        ],
    )
    def gather_kernel(pk, ids, out, pidx, off, slab, cc, sg, sw):
        wid = lax.axis_index("s") * NC + lax.axis_index("c")
        base = wid * bw
        iota = lax.iota(jnp.int32, 16)

        for c in range(nch):
            pltpu.sync_copy(ids.at[pl.ds(base + c * _CH, _CH)], pidx.at[c])

        qb = _Q.bit_length() - 1
        bb = qb + 2

        def idx_body(s, c):
            sl = pl.ds(s * 16, 16)
            u = pidx[c, sl]
            off[c, sl] = ((u >> qb) & 3) << 5
            pidx[c, sl] = ((u >> bb) << qb) | (u & (_Q - 1))
            return c

        for c in range(nch):
            lax.fori_loop(0, _CH // 16, idx_body, c)

        def zero_body(r, _):
            z = jnp.zeros((16,), jnp.float32)
            for b in range(2):
                for l0 in range(32, 128, 16):
                    cc[b, r, pl.ds(l0, 16)] = z
            return 0

        lax.fori_loop(0, _CH, zero_body, 0)

        def issue(c):
            return pltpu.async_copy(pk.at[pidx.at[c]], slab.at[c % 2], sg)

        def extract(c):
            b = c % 2

            def ex_body(s, _):
                rows = s * 16 + iota
                o = off[c, pl.ds(s * 16, 16)]
                for d in range(32):
                    lane = jnp.full((16,), d, jnp.int32)
                    val = plsc.load_gather(slab.at[b], [rows, o + d])
                    plsc.store_scatter(cc.at[b], [rows, lane], val)
                return 0

            lax.fori_loop(0, _CH // 16, ex_body, 0)

        pend = {0: issue(0)}
        writes = []
        for c in range(nch):
            if c + 1 < nch:
                pend[c + 1] = issue(c + 1)
            pend.pop(c).wait()
            if c >= 2:
                writes[c - 2].wait()
            extract(c)
            writes.append(pltpu.async_copy(
                cc.at[c % 2], out.at[pl.ds(base + c * _CH, _CH)], sw))
        writes[-2].wait()
        writes[-1].wait()

    return gather_kernel


def _mlp_body(u_ref, v_ref, wu_ref, wv_ref, b1_ref, wo_ref, bo_ref, o_ref):
    h = jnp.dot(u_ref[...], wu_ref[...], preferred_element_type=jnp.float32)
    h = h + jnp.dot(v_ref[...], wv_ref[...], preferred_element_type=jnp.float32)
    h = jnp.maximum(h + b1_ref[...], 0.0)
    o_ref[...] = jnp.dot(h, wo_ref[...], preferred_element_type=jnp.float32) + bo_ref[...]


@functools.lru_cache(maxsize=None)
def _make_mlp(B, H, O):
    blk = 2048
    return pl.pallas_call(
        _mlp_body,
        grid=(B // blk,),
        in_specs=[
            pl.BlockSpec((blk, 128), lambda i: (i, 0)),
            pl.BlockSpec((blk, 128), lambda i: (i, 0)),
            pl.BlockSpec((128, H), lambda i: (0, 0)),
            pl.BlockSpec((128, H), lambda i: (0, 0)),
            pl.BlockSpec((1, H), lambda i: (0, 0)),
            pl.BlockSpec((H, O), lambda i: (0, 0)),
            pl.BlockSpec((1, O), lambda i: (0, 0)),
        ],
        out_specs=pl.BlockSpec((blk, O), lambda i: (i, 0)),
        out_shape=jax.ShapeDtypeStruct((B, O), jnp.float32),
    )


def kernel(user_id, video_id, user_table, video_table, W1, b1, Wout, bout):
    B = user_id.shape[0]
    NU, D = user_table.shape
    NV = video_table.shape[0]
    vpk = _make_pack(NV, D)(video_table.T)
    ut, vpk = lax.optimization_barrier((user_table, vpk))
    ccv = _make_sc_gather(B, vpk.shape[0] // _Q)(vpk, video_id.astype(jnp.int32))
    upk = _make_pack(NU, D)(ut.T)
    ccu = _make_sc_gather(B, upk.shape[0] // _Q)(upk, user_id.astype(jnp.int32))
    H = W1.shape[0]
    O = Wout.shape[0]
    wu = jnp.zeros((128, H), jnp.float32).at[:D].set(W1[:, :D].T)
    wv = jnp.zeros((128, H), jnp.float32).at[:D].set(W1[:, D:].T)
    return _make_mlp(B, H, O)(ccu, ccv, wu, wv, b1[None, :], Wout.T, bout[None, :])


# speedup vs baseline: 3.8502x; 1.0440x over previous
import functools
import math

import jax
import jax.numpy as jnp
from jax import lax
from jax.experimental import pallas as pl
from jax.experimental.pallas import tpu as pltpu
from jax.experimental.pallas import tpu_sc as plsc

_BLKU = 65536
_Q = _BLKU // 4
_CH = 128


def _pack_body(in_ref, o_ref):
    x = in_ref[...]
    d = x.shape[0]
    rows = lax.broadcasted_iota(jnp.int32, (d, 4 * d), 0)
    cols = lax.broadcasted_iota(jnp.int32, (d, 4 * d), 1)
    acc = None
    for j in range(4):
        xj = x[:, j * _Q:(j + 1) * _Q]
        ej = jnp.where(cols == rows + d * j, 1.0, 0.0).astype(jnp.bfloat16)
        y = lax.dot_general(xj.astype(jnp.bfloat16), ej,
                            (((0,), (0,)), ((), ())),
                            preferred_element_type=jnp.float32)
        acc = y if acc is None else acc + y
    o_ref[...] = acc


@functools.lru_cache(maxsize=None)
def _make_pack(N, D):
    g = math.ceil(N / _BLKU)
    return pl.pallas_call(
        _pack_body,
        grid=(g,),
        in_specs=[pl.BlockSpec((D, _BLKU), lambda i: (0, i))],
        out_specs=pl.BlockSpec((_Q, 128), lambda i: (i, 0)),
        out_shape=jax.ShapeDtypeStruct((g * _Q, 128), jnp.float32),
    )


@functools.lru_cache(maxsize=None)
def _make_sc_gather(B, G):
    info = plsc.get_sparse_core_info()
    NC, NS = info.num_cores, info.num_subcores
    NW = NC * NS
    bw = B // NW
    nch = bw // _CH
    mesh = plsc.VectorSubcoreMesh(core_axis_name="c", subcore_axis_name="s")

    @functools.partial(
        pl.kernel,
        mesh=mesh,
        compiler_params=pltpu.CompilerParams(
            use_tc_tiling_on_sc=False, needs_layout_passes=False),
        out_type=jax.ShapeDtypeStruct((B, 128), jnp.float32),
        scratch_types=[
            pltpu.VMEM((nch, _CH), jnp.int32),
            pltpu.VMEM((nch, _CH), jnp.int32),
            pltpu.VMEM((2, _CH, 128), jnp.float32),
            pltpu.VMEM((2, _CH, 128), jnp.float32),
            pltpu.SemaphoreType.DMA,
            pltpu.SemaphoreType.DMA,
        ],
    )
    def gather_kernel(pk, ids, out, pidx, off, slab, cc, sg, sw):
        wid = lax.axis_index("s") * NC + lax.axis_index("c")
        base = wid * bw
        iota = lax.iota(jnp.int32, 16)

        for c in range(nch):
            pltpu.sync_copy(ids.at[pl.ds(base + c * _CH, _CH)], pidx.at[c])

        qb = _Q.bit_length() - 1
        bb = qb + 2

        def idx_body(s, c):
            sl = pl.ds(s * 16, 16)
            u = pidx[c, sl]
            off[c, sl] = ((u >> qb) & 3) << 5
            pidx[c, sl] = ((u >> bb) << qb) | (u & (_Q - 1))
            return c

        for c in range(nch):
            lax.fori_loop(0, _CH // 16, idx_body, c)

        def zero_body(r, _):
            z = jnp.zeros((16,), jnp.float32)
            for b in range(2):
                for l0 in range(32, 128, 16):
                    cc[b, r, pl.ds(l0, 16)] = z
            return 0

        lax.fori_loop(0, _CH, zero_body, 0)

        def issue(c):
            return pltpu.async_copy(pk.at[pidx.at[c]], slab.at[c % 2], sg)

        def extract(c):
            b = c % 2

            def ex_body(s, _):
                rows = s * 16 + iota
                o = off[c, pl.ds(s * 16, 16)]
                for d in range(32):
                    lane = jnp.full((16,), d, jnp.int32)
                    val = plsc.load_gather(slab.at[b], [rows, o + d])
                    plsc.store_scatter(cc.at[b], [rows, lane], val)
                return 0

            lax.fori_loop(0, _CH // 16, ex_body, 0)

        pend = {0: issue(0)}
        writes = []
        for c in range(nch):
            if c + 1 < nch:
                pend[c + 1] = issue(c + 1)
            pend.pop(c).wait()
            if c >= 2:
                writes[c - 2].wait()
            extract(c)
            writes.append(pltpu.async_copy(
                cc.at[c % 2], out.at[pl.ds(base + c * _CH, _CH)], sw))
        writes[-2].wait()
        writes[-1].wait()

    return gather_kernel


def _mlp_body(u_ref, v_ref, wu_ref, wv_ref, b1_ref, wo_ref, bo_ref, o_ref):
    h = jnp.dot(u_ref[...], wu_ref[...], preferred_element_type=jnp.float32)
    h = h + jnp.dot(v_ref[...], wv_ref[...], preferred_element_type=jnp.float32)
    h = jnp.maximum(h + b1_ref[...], 0.0)
    o_ref[...] = jnp.dot(h, wo_ref[...], preferred_element_type=jnp.float32) + bo_ref[...]


@functools.lru_cache(maxsize=None)
def _make_mlp(B, H, O):
    blk = 2048
    return pl.pallas_call(
        _mlp_body,
        grid=(B // blk,),
        in_specs=[
            pl.BlockSpec((blk, 128), lambda i: (i, 0)),
            pl.BlockSpec((blk, 128), lambda i: (i, 0)),
            pl.BlockSpec((128, H), lambda i: (0, 0)),
            pl.BlockSpec((128, H), lambda i: (0, 0)),
            pl.BlockSpec((1, H), lambda i: (0, 0)),
            pl.BlockSpec((H, O), lambda i: (0, 0)),
            pl.BlockSpec((1, O), lambda i: (0, 0)),
        ],
        out_specs=pl.BlockSpec((blk, O), lambda i: (i, 0)),
        out_shape=jax.ShapeDtypeStruct((B, O), jnp.float32),
    )


def kernel(user_id, video_id, user_table, video_table, W1, b1, Wout, bout):
    B = user_id.shape[0]
    NU, D = user_table.shape
    NV = video_table.shape[0]
    vpk = _make_pack(NV, D)(video_table.T)
    ut, vpk = lax.optimization_barrier((user_table, vpk))
    ccv = _make_sc_gather(B, vpk.shape[0] // _Q)(vpk, video_id.astype(jnp.int32))
    upk = _make_pack(NU, D)(ut.T)
    ccu = _make_sc_gather(B, upk.shape[0] // _Q)(upk, user_id.astype(jnp.int32))
    H = W1.shape[0]
    O = Wout.shape[0]
    wu = jnp.zeros((128, H), jnp.float32).at[:D].set(W1[:, :D].T)
    wv = jnp.zeros((128, H), jnp.float32).at[:D].set(W1[:, D:].T)
    return _make_mlp(B, H, O)(ccu, ccv, wu, wv, b1[None, :], Wout.T, bout[None, :])
